# baseline (speedup 1.0000x reference)
import functools

import numpy as np
import jax
import jax.numpy as jnp
from jax import lax
from jax.experimental import pallas as pl
from jax.experimental.pallas import tpu as pltpu
from jax.experimental.pallas import tpu_sc as plsc

ALPHA = 0.1
THETA = 0.5
NUM_LAYERS = 8
N_NODES = 10000
N_EDGES = 320000
D = 128

NC = 2
NS = 16
NW = NC * NS
HALF_N = N_NODES // 2
ACC_ROWS = 5120
TRASH = HALF_N

EDGES_PER_TILE = 10240
E_PAD = NW * EDGES_PER_TILE
CELL = 5632
SLOTS_PER_TILE = 2 * CELL
CHUNK = 128
NCH = SLOTS_PER_TILE // CHUNK
NBUF = 2

ZROWS_PER_TILE = ACC_ROWS // NS
OUT_ROWS_PER_TILE = 312

_mesh = plsc.VectorSubcoreMesh(core_axis_name="c", subcore_axis_name="s")


def _router_body(src_hbm, dst_hbm, osrc_hbm, odst_hbm,
                 src_v, dst_v, sb0, db0, sb1, db1):
    cid = lax.axis_index("c")
    sid = lax.axis_index("s")
    wid = sid * NC + cid

    pltpu.sync_copy(src_hbm.at[wid], src_v)
    pltpu.sync_copy(dst_hbm.at[wid], dst_v)

    zsrc = jnp.zeros((16,), jnp.int32)
    ztrash = jnp.full((16,), TRASH, jnp.int32)

    def _fill(i, carry):
        for buf, val in ((sb0, zsrc), (db0, ztrash), (sb1, zsrc), (db1, ztrash)):
            buf[pl.ds(i * 16, 16)] = val
        return carry
    lax.fori_loop(0, CELL // 16, _fill, 0)

    def _route(g, carry):
        c0, c1 = carry
        s16 = src_v[pl.ds(g * 16, 16)]
        d16 = dst_v[pl.ds(g * 16, 16)]
        m0 = d16 < HALF_N
        m1 = jnp.logical_not(m0)
        cs0 = plsc.cumsum(m0.astype(jnp.int32))
        cs1 = plsc.cumsum(m1.astype(jnp.int32))
        pos0 = c0 + cs0 - 1
        pos1 = c1 + cs1 - 1
        plsc.store_scatter(sb0, [pos0], s16, mask=m0)
        plsc.store_scatter(db0, [pos0], d16, mask=m0)
        plsc.store_scatter(sb1, [pos1], s16, mask=m1)
        plsc.store_scatter(db1, [pos1], d16 - HALF_N, mask=m1)
        n0 = jnp.sum(m0.astype(jnp.int32))
        return c0 + n0, c1 + (16 - n0)
    lax.fori_loop(0, EDGES_PER_TILE // 16, _route, (jnp.int32(0), jnp.int32(0)))

    s_out = wid // 2
    off = (wid % 2) * CELL
    pltpu.sync_copy(sb0, osrc_hbm.at[0, s_out, pl.ds(off, CELL)])
    pltpu.sync_copy(db0, odst_hbm.at[0, s_out, pl.ds(off, CELL)])
    pltpu.sync_copy(sb1, osrc_hbm.at[1, s_out, pl.ds(off, CELL)])
    pltpu.sync_copy(db1, odst_hbm.at[1, s_out, pl.ds(off, CELL)])


_router = pl.kernel(
    _router_body,
    out_type=(jax.ShapeDtypeStruct((NC, NS, SLOTS_PER_TILE), jnp.int32),
              jax.ShapeDtypeStruct((NC, NS, SLOTS_PER_TILE), jnp.int32)),
    mesh=_mesh,
    scratch_types=[
        pltpu.VMEM((EDGES_PER_TILE,), jnp.int32),
        pltpu.VMEM((EDGES_PER_TILE,), jnp.int32),
        pltpu.VMEM((CELL,), jnp.int32),
        pltpu.VMEM((CELL,), jnp.int32),
        pltpu.VMEM((CELL,), jnp.int32),
        pltpu.VMEM((CELL,), jnp.int32),
    ],
    compiler_params=pltpu.CompilerParams(needs_layout_passes=False),
)


def _agg_body(h_hbm, src_hbm, dst_hbm, out_hbm,
              src_v, dst_v, zbuf, acc, *gbufs_sems):
    gbufs = gbufs_sems[:NBUF]
    sems = gbufs_sems[NBUF:]
    cid = lax.axis_index("c")
    sid = lax.axis_index("s")

    pltpu.sync_copy(src_hbm.at[cid, sid], src_v)
    pltpu.sync_copy(dst_hbm.at[cid, sid], dst_v)

    zero16 = jnp.zeros((16,), jnp.float32)
    for r in range(8):
        for c8 in range(D // 16):
            zbuf[r, pl.ds(c8 * 16, 16)] = zero16

    def _zero(i, carry):
        pltpu.sync_copy(zbuf, acc.at[pl.ds(sid * ZROWS_PER_TILE + i * 8, 8)])
        return carry
    lax.fori_loop(0, ZROWS_PER_TILE // 8, _zero, 0)

    plsc.subcore_barrier()

    for b in range(NBUF):
        pltpu.async_copy(h_hbm.at[src_v.at[b]], gbufs[b], sems[b])

    def _body(g, carry):
        i0 = g * NBUF
        for b in range(NBUF):
            i = i0 + b
            pltpu.make_async_copy(h_hbm.at[src_v.at[i]], gbufs[b],
                                  sems[b]).wait()
            pltpu.sync_copy(gbufs[b], acc.at[dst_v.at[i]], add=True)

            @pl.when(i + NBUF < NCH)
            def _prefetch(i=i, b=b):
                pltpu.async_copy(h_hbm.at[src_v.at[i + NBUF]], gbufs[b],
                                 sems[b])
        return carry
    lax.fori_loop(0, NCH // NBUF, _body, 0)

    plsc.subcore_barrier()

    pltpu.sync_copy(
        acc.at[pl.ds(sid * OUT_ROWS_PER_TILE, OUT_ROWS_PER_TILE)],
        out_hbm.at[pl.ds(cid * HALF_N + sid * OUT_ROWS_PER_TILE,
                         OUT_ROWS_PER_TILE)])

    tail = NS * OUT_ROWS_PER_TILE

    @pl.when(sid == NS - 1)
    def _tail():
        pltpu.sync_copy(
            acc.at[pl.ds(tail, HALF_N - tail)],
            out_hbm.at[pl.ds(cid * HALF_N + tail, HALF_N - tail)])


_agg = pl.kernel(
    _agg_body,
    out_type=jax.ShapeDtypeStruct((N_NODES, D), jnp.float32),
    mesh=_mesh,
    scratch_types=(
        [pltpu.VMEM((NCH, CHUNK), jnp.int32),
         pltpu.VMEM((NCH, CHUNK), jnp.int32),
         pltpu.VMEM((8, D), jnp.float32),
         pltpu.VMEM_SHARED((ACC_ROWS, D), jnp.float32)]
        + [pltpu.VMEM((CHUNK, D), jnp.float32) for _ in range(NBUF)]
        + [pltpu.SemaphoreType.DMA for _ in range(NBUF)]
    ),
)


_ROW_BLOCK = 2000


def _mm_bias_body(x_ref, w_ref, b_ref, o_ref, *, relu):
    acc = jnp.dot(x_ref[...], w_ref[...], preferred_element_type=jnp.float32,
                  precision=lax.Precision.HIGHEST)
    acc = acc + b_ref[...]
    if relu:
        acc = jnp.maximum(acc, 0.0)
    o_ref[...] = acc


def _mm_bias(x, w, b, relu):
    return pl.pallas_call(
        functools.partial(_mm_bias_body, relu=relu),
        grid=(N_NODES // _ROW_BLOCK,),
        in_specs=[pl.BlockSpec((_ROW_BLOCK, D), lambda i: (i, 0)),
                  pl.BlockSpec((D, D), lambda i: (0, 0)),
                  pl.BlockSpec((1, D), lambda i: (0, 0))],
        out_specs=pl.BlockSpec((_ROW_BLOCK, D), lambda i: (i, 0)),
        out_shape=jax.ShapeDtypeStruct((N_NODES, D), jnp.float32),
    )(x, w, b.reshape(1, D))


def _layer_body(agg_ref, x0_ref, w_ref, o_ref, *, beta):
    hx = (1.0 - ALPHA) * agg_ref[...] + ALPHA * x0_ref[...]
    mm = jnp.dot(hx, w_ref[...], preferred_element_type=jnp.float32,
                 precision=lax.Precision.HIGHEST)
    o_ref[...] = jnp.maximum((1.0 - beta) * hx + beta * mm, 0.0)


def _layer_update(agg, x0, w, beta):
    return pl.pallas_call(
        functools.partial(_layer_body, beta=beta),
        grid=(N_NODES // _ROW_BLOCK,),
        in_specs=[pl.BlockSpec((_ROW_BLOCK, D), lambda i: (i, 0)),
                  pl.BlockSpec((_ROW_BLOCK, D), lambda i: (i, 0)),
                  pl.BlockSpec((D, D), lambda i: (0, 0))],
        out_specs=pl.BlockSpec((_ROW_BLOCK, D), lambda i: (i, 0)),
        out_shape=jax.ShapeDtypeStruct((N_NODES, D), jnp.float32),
    )(agg, x0, w)


def kernel(x, edge_index, W_in, b_in, W_out, b_out, Ws):
    src = edge_index[0].astype(jnp.int32)
    dst = edge_index[1].astype(jnp.int32)
    pad = E_PAD - N_EDGES
    src_p = jnp.concatenate([src, jnp.zeros((pad,), jnp.int32)])
    dst_p = jnp.concatenate([dst, jnp.full((pad,), N_NODES, jnp.int32)])
    src_p = jnp.transpose(src_p.reshape(EDGES_PER_TILE, NW))
    dst_p = jnp.transpose(dst_p.reshape(EDGES_PER_TILE, NW))

    rsrc, rdst = _router(src_p, dst_p)
    rsrc = rsrc.reshape(NC, NS, NCH, CHUNK)
    rdst = rdst.reshape(NC, NS, NCH, CHUNK)

    h = _mm_bias(x, W_in, b_in, relu=True)
    x0 = h
    for layer in range(NUM_LAYERS):
        beta = float(np.log(THETA / (layer + 1) + 1.0))
        agg = _agg(h, rsrc, rdst)
        h = _layer_update(agg, x0, w=Ws[layer], beta=beta)
    return _mm_bias(h, W_out, b_out, relu=False)


# speedup vs baseline: 12.5823x; 12.5823x over previous
import functools

import numpy as np
import jax
import jax.numpy as jnp
from jax import lax
from jax.experimental import pallas as pl
from jax.experimental.pallas import tpu as pltpu
from jax.experimental.pallas import tpu_sc as plsc

ALPHA = 0.1
THETA = 0.5
NUM_LAYERS = 8
N_NODES = 10000
N_EDGES = 320000
D = 128

NC = 2
NS = 16
NW = NC * NS
HALF_N = N_NODES // 2
ACC_ROWS = 5120
TRASH = HALF_N

EDGES_PER_TILE = 10240
E_PAD = NW * EDGES_PER_TILE
CELL = 5632
SLOTS_PER_TILE = 2 * CELL
CHUNK = 128
NCH = SLOTS_PER_TILE // CHUNK
NBUF = 2

ZROWS_PER_TILE = ACC_ROWS // NS
OUT_ROWS_PER_TILE = 312

_mesh = plsc.VectorSubcoreMesh(core_axis_name="c", subcore_axis_name="s")


def _router_body(src_hbm, dst_hbm, osrc_hbm, odst_hbm,
                 src_v, dst_v, sb0, db0, sb1, db1):
    cid = lax.axis_index("c")
    sid = lax.axis_index("s")
    wid = sid * NC + cid

    pltpu.sync_copy(src_hbm.at[wid], src_v)
    pltpu.sync_copy(dst_hbm.at[wid], dst_v)

    iota16 = plsc.cumsum(jnp.ones((16,), jnp.int32)) - 1

    def _fill(i, carry):
        tsrc = (i * 16 + iota16 + wid * 613) & 8191
        tdst = TRASH + ((i * 16 + iota16 + (wid // 2) * 4) & 63)
        for buf, val in ((sb0, tsrc), (db0, tdst), (sb1, tsrc), (db1, tdst)):
            buf[pl.ds(i * 16, 16)] = val
        return carry
    lax.fori_loop(0, CELL // 16, _fill, 0)

    def _route(g, carry):
        c0, c1 = carry
        s16 = src_v[pl.ds(g * 16, 16)]
        d16 = dst_v[pl.ds(g * 16, 16)]
        m0 = d16 < HALF_N
        m1 = jnp.logical_not(m0)
        cs0 = plsc.cumsum(m0.astype(jnp.int32))
        cs1 = plsc.cumsum(m1.astype(jnp.int32))
        pos0 = c0 + cs0 - 1
        pos1 = c1 + cs1 - 1
        plsc.store_scatter(sb0, [pos0], s16, mask=m0)
        plsc.store_scatter(db0, [pos0], d16, mask=m0)
        plsc.store_scatter(sb1, [pos1], s16, mask=m1)
        plsc.store_scatter(db1, [pos1], d16 - HALF_N, mask=m1)
        n0 = jnp.sum(m0.astype(jnp.int32))
        return c0 + n0, c1 + (16 - n0)
    lax.fori_loop(0, EDGES_PER_TILE // 16, _route, (jnp.int32(0), jnp.int32(0)))

    s_out = wid // 2
    off = (wid % 2) * CELL
    pltpu.sync_copy(sb0, osrc_hbm.at[0, s_out, pl.ds(off, CELL)])
    pltpu.sync_copy(db0, odst_hbm.at[0, s_out, pl.ds(off, CELL)])
    pltpu.sync_copy(sb1, osrc_hbm.at[1, s_out, pl.ds(off, CELL)])
    pltpu.sync_copy(db1, odst_hbm.at[1, s_out, pl.ds(off, CELL)])


_router = pl.kernel(
    _router_body,
    out_type=(jax.ShapeDtypeStruct((NC, NS, SLOTS_PER_TILE), jnp.int32),
              jax.ShapeDtypeStruct((NC, NS, SLOTS_PER_TILE), jnp.int32)),
    mesh=_mesh,
    scratch_types=[
        pltpu.VMEM((EDGES_PER_TILE,), jnp.int32),
        pltpu.VMEM((EDGES_PER_TILE,), jnp.int32),
        pltpu.VMEM((CELL,), jnp.int32),
        pltpu.VMEM((CELL,), jnp.int32),
        pltpu.VMEM((CELL,), jnp.int32),
        pltpu.VMEM((CELL,), jnp.int32),
    ],
    compiler_params=pltpu.CompilerParams(needs_layout_passes=False),
)


def _agg_body(h_hbm, src_hbm, dst_hbm, out_hbm,
              src_v, dst_v, zbuf, acc, *gbufs_sems):
    gbufs = gbufs_sems[:NBUF]
    sems = gbufs_sems[NBUF:]
    cid = lax.axis_index("c")
    sid = lax.axis_index("s")

    pltpu.sync_copy(src_hbm.at[cid, sid], src_v)
    pltpu.sync_copy(dst_hbm.at[cid, sid], dst_v)

    zero16 = jnp.zeros((16,), jnp.float32)
    for r in range(8):
        for c8 in range(D // 16):
            zbuf[r, pl.ds(c8 * 16, 16)] = zero16

    def _zero(i, carry):
        pltpu.sync_copy(zbuf, acc.at[pl.ds(sid * ZROWS_PER_TILE + i * 8, 8)])
        return carry
    lax.fori_loop(0, ZROWS_PER_TILE // 8, _zero, 0)

    plsc.subcore_barrier()

    for b in range(NBUF):
        pltpu.async_copy(h_hbm.at[src_v.at[b]], gbufs[b], sems[b])

    def _body(g, carry):
        i0 = g * NBUF
        for b in range(NBUF):
            i = i0 + b
            pltpu.make_async_copy(h_hbm.at[src_v.at[i]], gbufs[b],
                                  sems[b]).wait()
            pltpu.sync_copy(gbufs[b], acc.at[dst_v.at[i]], add=True)

            @pl.when(i + NBUF < NCH)
            def _prefetch(i=i, b=b):
                pltpu.async_copy(h_hbm.at[src_v.at[i + NBUF]], gbufs[b],
                                 sems[b])
        return carry
    lax.fori_loop(0, NCH // NBUF, _body, 0)

    plsc.subcore_barrier()

    pltpu.sync_copy(
        acc.at[pl.ds(sid * OUT_ROWS_PER_TILE, OUT_ROWS_PER_TILE)],
        out_hbm.at[pl.ds(cid * HALF_N + sid * OUT_ROWS_PER_TILE,
                         OUT_ROWS_PER_TILE)])

    tail = NS * OUT_ROWS_PER_TILE

    @pl.when(sid == NS - 1)
    def _tail():
        pltpu.sync_copy(
            acc.at[pl.ds(tail, HALF_N - tail)],
            out_hbm.at[pl.ds(cid * HALF_N + tail, HALF_N - tail)])


_agg = pl.kernel(
    _agg_body,
    out_type=jax.ShapeDtypeStruct((N_NODES, D), jnp.float32),
    mesh=_mesh,
    scratch_types=(
        [pltpu.VMEM((NCH, CHUNK), jnp.int32),
         pltpu.VMEM((NCH, CHUNK), jnp.int32),
         pltpu.VMEM((8, D), jnp.float32),
         pltpu.VMEM_SHARED((ACC_ROWS, D), jnp.float32)]
        + [pltpu.VMEM((CHUNK, D), jnp.float32) for _ in range(NBUF)]
        + [pltpu.SemaphoreType.DMA for _ in range(NBUF)]
    ),
)


_ROW_BLOCK = 2000


def _mm_bias_body(x_ref, w_ref, b_ref, o_ref, *, relu):
    acc = jnp.dot(x_ref[...], w_ref[...], preferred_element_type=jnp.float32,
                  precision=lax.Precision.HIGHEST)
    acc = acc + b_ref[...]
    if relu:
        acc = jnp.maximum(acc, 0.0)
    o_ref[...] = acc


def _mm_bias(x, w, b, relu):
    return pl.pallas_call(
        functools.partial(_mm_bias_body, relu=relu),
        grid=(N_NODES // _ROW_BLOCK,),
        in_specs=[pl.BlockSpec((_ROW_BLOCK, D), lambda i: (i, 0)),
                  pl.BlockSpec((D, D), lambda i: (0, 0)),
                  pl.BlockSpec((1, D), lambda i: (0, 0))],
        out_specs=pl.BlockSpec((_ROW_BLOCK, D), lambda i: (i, 0)),
        out_shape=jax.ShapeDtypeStruct((N_NODES, D), jnp.float32),
    )(x, w, b.reshape(1, D))


def _layer_body(agg_ref, x0_ref, w_ref, o_ref, *, beta):
    hx = (1.0 - ALPHA) * agg_ref[...] + ALPHA * x0_ref[...]
    mm = jnp.dot(hx, w_ref[...], preferred_element_type=jnp.float32,
                 precision=lax.Precision.HIGHEST)
    o_ref[...] = jnp.maximum((1.0 - beta) * hx + beta * mm, 0.0)


def _layer_update(agg, x0, w, beta):
    return pl.pallas_call(
        functools.partial(_layer_body, beta=beta),
        grid=(N_NODES // _ROW_BLOCK,),
        in_specs=[pl.BlockSpec((_ROW_BLOCK, D), lambda i: (i, 0)),
                  pl.BlockSpec((_ROW_BLOCK, D), lambda i: (i, 0)),
                  pl.BlockSpec((D, D), lambda i: (0, 0))],
        out_specs=pl.BlockSpec((_ROW_BLOCK, D), lambda i: (i, 0)),
        out_shape=jax.ShapeDtypeStruct((N_NODES, D), jnp.float32),
    )(agg, x0, w)


def kernel(x, edge_index, W_in, b_in, W_out, b_out, Ws):
    src = edge_index[0].astype(jnp.int32)
    dst = edge_index[1].astype(jnp.int32)
    pad = E_PAD - N_EDGES
    pidx = jnp.arange(pad, dtype=jnp.int32)
    src_p = jnp.concatenate([src, pidx & 8191])
    dst_p = jnp.concatenate([dst, N_NODES + (pidx & 63)])
    src_p = jnp.transpose(src_p.reshape(EDGES_PER_TILE, NW))
    dst_p = jnp.transpose(dst_p.reshape(EDGES_PER_TILE, NW))

    rsrc, rdst = _router(src_p, dst_p)
    rsrc = rsrc.reshape(NC, NS, NCH, CHUNK)
    rdst = rdst.reshape(NC, NS, NCH, CHUNK)

    h = _mm_bias(x, W_in, b_in, relu=True)
    x0 = h
    for layer in range(NUM_LAYERS):
        beta = float(np.log(THETA / (layer + 1) + 1.0))
        agg = _agg(h, rsrc, rdst)
        h = _layer_update(agg, x0, w=Ws[layer], beta=beta)
    return _mm_bias(h, W_out, b_out, relu=False)
